# baseline (speedup 1.0000x reference)
import functools
import math

import jax
import jax.numpy as jnp
from jax.experimental import pallas as pl
from jax.experimental.pallas import tpu as pltpu

_BASE_GSD = 10.0
_LN10K = math.log(10000.0)
_MONTH_SCALE = 2.0 * math.pi / 12.0


def _tables_kernel(months_ref, gsd_ref, ce_ref, a_ref, s_ref, *, w_cnt):
    bi = pl.program_id(0)
    ts, d = a_ref.shape[1], a_ref.shape[2]
    n = d // 4
    half = n // 2
    s_cnt = ce_ref.shape[0]
    t_cnt = ts // s_cnt

    r = jax.lax.broadcasted_iota(jnp.int32, (ts, half), 0)
    lane = jax.lax.broadcasted_iota(jnp.int32, (ts, half), 1).astype(jnp.float32)
    t_idx = r // s_cnt
    omega = jnp.exp(lane * (-_LN10K / half))
    arg = t_idx.astype(jnp.float32) * omega
    q1 = jnp.concatenate([jnp.sin(arg), jnp.cos(arg)], axis=1)

    m = jnp.zeros((ts, half), jnp.float32)
    for j in range(t_cnt):
        mj = months_ref[bi, j].astype(jnp.float32)
        m = jnp.where(t_idx == j, mj, m)
    ang = m * _MONTH_SCALE
    q2 = jnp.concatenate([jnp.sin(ang), jnp.cos(ang)], axis=1)

    s_idx = jax.lax.broadcasted_iota(jnp.int32, (ts, n), 0) % s_cnt
    q0 = jnp.zeros((ts, n), jnp.float32)
    for j in range(s_cnt):
        q0 = jnp.where(s_idx == j, ce_ref[j, :][None, :], q0)

    a_ref[0] = jnp.concatenate(
        [q0, q1, q2, jnp.zeros((ts, n), jnp.float32)], axis=1)

    hw = s_ref.shape[0]
    quarter = half // 2
    g = jax.lax.broadcasted_iota(jnp.int32, (hw, quarter), 0)
    lane_q = jax.lax.broadcasted_iota(jnp.int32, (hw, quarter), 1).astype(jnp.float32)
    gsd = gsd_ref[0, 0]
    omega_q = jnp.exp(lane_q * (-_LN10K / quarter))
    ph = (g // w_cnt).astype(jnp.float32) * gsd
    pw = (g % w_cnt).astype(jnp.float32) * gsd
    argh = ph * omega_q
    argw = pw * omega_q
    q3 = jnp.concatenate(
        [jnp.sin(argh), jnp.cos(argh), jnp.sin(argw), jnp.cos(argw)], axis=1)
    s_ref[...] = jnp.concatenate(
        [jnp.zeros((hw, 3 * n), jnp.float32), q3], axis=1)


def _stream_kernel(tok_ref, a_ref, s_ref, out_ref):
    out_ref[...] = tok_ref[...] + a_ref[...] + s_ref[...][:, None, :]


def kernel(tokens, channel_embed, timestamps, patch_size, input_res):
    b, h, w, t, s, d = tokens.shape
    n = d // 4
    ts = t * s
    hw = h * w

    months = timestamps[:, :, 1].astype(jnp.int32)
    gsd = (jnp.asarray(input_res, jnp.float32)
           * jnp.asarray(patch_size, jnp.float32) / _BASE_GSD).reshape(1, 1)

    a_tab, s_tab = pl.pallas_call(
        functools.partial(_tables_kernel, w_cnt=w),
        grid=(b,),
        in_specs=[
            pl.BlockSpec(memory_space=pltpu.SMEM),
            pl.BlockSpec(memory_space=pltpu.SMEM),
            pl.BlockSpec((channel_embed.shape[0], n), lambda i: (0, 0)),
        ],
        out_specs=[
            pl.BlockSpec((1, ts, d), lambda i: (i, 0, 0)),
            pl.BlockSpec((hw, d), lambda i: (0, 0)),
        ],
        out_shape=[
            jax.ShapeDtypeStruct((b, ts, d), jnp.float32),
            jax.ShapeDtypeStruct((hw, d), jnp.float32),
        ],
    )(months, gsd, channel_embed)

    hw_blk = 8
    blocks_per_b = hw // hw_blk
    tok_r = tokens.reshape(b * hw, ts, d)
    out = pl.pallas_call(
        _stream_kernel,
        grid=(b * blocks_per_b,),
        in_specs=[
            pl.BlockSpec((hw_blk, ts, d), lambda i: (i, 0, 0)),
            pl.BlockSpec((1, ts, d), lambda i: (i // blocks_per_b, 0, 0)),
            pl.BlockSpec((hw_blk, d), lambda i: (i % blocks_per_b, 0)),
        ],
        out_specs=pl.BlockSpec((hw_blk, ts, d), lambda i: (i, 0, 0)),
        out_shape=jax.ShapeDtypeStruct((b * hw, ts, d), jnp.float32),
        compiler_params=pltpu.CompilerParams(
            dimension_semantics=("parallel",)),
    )(tok_r, a_tab, s_tab)
    return out.reshape(tokens.shape)


# speedup vs baseline: 1.0561x; 1.0561x over previous
import functools
import math

import jax
import jax.numpy as jnp
from jax.experimental import pallas as pl
from jax.experimental.pallas import tpu as pltpu

_BASE_GSD = 10.0
_LN10K = math.log(10000.0)
_MONTH_SCALE = 2.0 * math.pi / 12.0


def _tables_kernel(months_ref, gsd_ref, ce_ref, a_ref, s_ref, *, w_cnt):
    bi = pl.program_id(0)
    ts, d = a_ref.shape[1], a_ref.shape[2]
    n = d // 4
    half = n // 2
    s_cnt = ce_ref.shape[0]
    t_cnt = ts // s_cnt

    r = jax.lax.broadcasted_iota(jnp.int32, (ts, half), 0)
    lane = jax.lax.broadcasted_iota(jnp.int32, (ts, half), 1).astype(jnp.float32)
    t_idx = r // s_cnt
    omega = jnp.exp(lane * (-_LN10K / half))
    arg = t_idx.astype(jnp.float32) * omega
    q1 = jnp.concatenate([jnp.sin(arg), jnp.cos(arg)], axis=1)

    m = jnp.zeros((ts, half), jnp.float32)
    for j in range(t_cnt):
        mj = months_ref[bi, j].astype(jnp.float32)
        m = jnp.where(t_idx == j, mj, m)
    ang = m * _MONTH_SCALE
    q2 = jnp.concatenate([jnp.sin(ang), jnp.cos(ang)], axis=1)

    s_idx = jax.lax.broadcasted_iota(jnp.int32, (ts, n), 0) % s_cnt
    q0 = jnp.zeros((ts, n), jnp.float32)
    for j in range(s_cnt):
        q0 = jnp.where(s_idx == j, ce_ref[j, :][None, :], q0)

    a_ref[0] = jnp.concatenate(
        [q0, q1, q2, jnp.zeros((ts, n), jnp.float32)], axis=1)

    hw = s_ref.shape[0]
    quarter = half // 2
    g = jax.lax.broadcasted_iota(jnp.int32, (hw, quarter), 0)
    lane_q = jax.lax.broadcasted_iota(jnp.int32, (hw, quarter), 1).astype(jnp.float32)
    gsd = gsd_ref[0, 0]
    omega_q = jnp.exp(lane_q * (-_LN10K / quarter))
    ph = (g // w_cnt).astype(jnp.float32) * gsd
    pw = (g % w_cnt).astype(jnp.float32) * gsd
    argh = ph * omega_q
    argw = pw * omega_q
    q3 = jnp.concatenate(
        [jnp.sin(argh), jnp.cos(argh), jnp.sin(argw), jnp.cos(argw)], axis=1)
    s_ref[...] = jnp.concatenate(
        [jnp.zeros((hw, 3 * n), jnp.float32), q3], axis=1)


def _stream_kernel(tok0_ref, tok1_ref, a_ref, s_ref, out_ref):
    half = tok0_ref.shape[0]
    add = a_ref[...]
    out_ref[:half] = tok0_ref[...] + add + s_ref[:half][:, None, :]
    out_ref[half:] = tok1_ref[...] + add + s_ref[half:][:, None, :]


def kernel(tokens, channel_embed, timestamps, patch_size, input_res):
    b, h, w, t, s, d = tokens.shape
    n = d // 4
    ts = t * s
    hw = h * w

    months = timestamps[:, :, 1].astype(jnp.int32)
    gsd = (jnp.asarray(input_res, jnp.float32)
           * jnp.asarray(patch_size, jnp.float32) / _BASE_GSD).reshape(1, 1)

    a_tab, s_tab = pl.pallas_call(
        functools.partial(_tables_kernel, w_cnt=w),
        grid=(b,),
        in_specs=[
            pl.BlockSpec(memory_space=pltpu.SMEM),
            pl.BlockSpec(memory_space=pltpu.SMEM),
            pl.BlockSpec((channel_embed.shape[0], n), lambda i: (0, 0)),
        ],
        out_specs=[
            pl.BlockSpec((1, ts, d), lambda i: (i, 0, 0)),
            pl.BlockSpec((hw, d), lambda i: (0, 0)),
        ],
        out_shape=[
            jax.ShapeDtypeStruct((b, ts, d), jnp.float32),
            jax.ShapeDtypeStruct((hw, d), jnp.float32),
        ],
    )(months, gsd, channel_embed)

    hw_blk = 32
    blocks_per_b = hw // hw_blk
    tok_r = tokens.reshape(b * hw, ts, d)
    out = pl.pallas_call(
        _stream_kernel,
        grid=(b * blocks_per_b,),
        in_specs=[
            pl.BlockSpec((hw_blk // 2, ts, d), lambda i: (2 * i, 0, 0)),
            pl.BlockSpec((hw_blk // 2, ts, d), lambda i: (2 * i + 1, 0, 0)),
            pl.BlockSpec((1, ts, d), lambda i: (i // blocks_per_b, 0, 0)),
            pl.BlockSpec((hw_blk, d), lambda i: (i % blocks_per_b, 0)),
        ],
        out_specs=pl.BlockSpec((hw_blk, ts, d), lambda i: (i, 0, 0)),
        out_shape=jax.ShapeDtypeStruct((b * hw, ts, d), jnp.float32),
        compiler_params=pltpu.CompilerParams(
            dimension_semantics=("parallel",)),
    )(tok_r, tok_r, a_tab, s_tab)
    return out.reshape(tokens.shape)


# speedup vs baseline: 3.9592x; 3.7488x over previous
import functools
import math

import jax
import jax.numpy as jnp
from jax.experimental import pallas as pl
from jax.experimental.pallas import tpu as pltpu

_BASE_GSD = 10.0
_LN10K = math.log(10000.0)
_MONTH_SCALE = 2.0 * math.pi / 12.0


def _tables_kernel(months_ref, gsd_ref, ce_ref, a_ref, s_ref, *, w_cnt):
    bi = pl.program_id(0)
    ts, d = a_ref.shape[1], a_ref.shape[2]
    n = d // 4
    half = n // 2
    s_cnt = ce_ref.shape[0]
    t_cnt = ts // s_cnt

    r = jax.lax.broadcasted_iota(jnp.int32, (ts, half), 0)
    lane = jax.lax.broadcasted_iota(jnp.int32, (ts, half), 1).astype(jnp.float32)
    t_idx = r // s_cnt
    omega = jnp.exp(lane * (-_LN10K / half))
    arg = t_idx.astype(jnp.float32) * omega
    q1 = jnp.concatenate([jnp.sin(arg), jnp.cos(arg)], axis=1)

    m = jnp.zeros((ts, half), jnp.float32)
    for j in range(t_cnt):
        mj = months_ref[bi, j].astype(jnp.float32)
        m = jnp.where(t_idx == j, mj, m)
    ang = m * _MONTH_SCALE
    q2 = jnp.concatenate([jnp.sin(ang), jnp.cos(ang)], axis=1)

    s_idx = jax.lax.broadcasted_iota(jnp.int32, (ts, n), 0) % s_cnt
    q0 = jnp.zeros((ts, n), jnp.float32)
    for j in range(s_cnt):
        q0 = jnp.where(s_idx == j, ce_ref[j, :][None, :], q0)

    a_ref[0] = jnp.concatenate(
        [q0, q1, q2, jnp.zeros((ts, n), jnp.float32)], axis=1)

    hw = s_ref.shape[0]
    quarter = half // 2
    g = jax.lax.broadcasted_iota(jnp.int32, (hw, quarter), 0)
    lane_q = jax.lax.broadcasted_iota(jnp.int32, (hw, quarter), 1).astype(jnp.float32)
    gsd = gsd_ref[0, 0]
    omega_q = jnp.exp(lane_q * (-_LN10K / quarter))
    ph = (g // w_cnt).astype(jnp.float32) * gsd
    pw = (g % w_cnt).astype(jnp.float32) * gsd
    argh = ph * omega_q
    argw = pw * omega_q
    q3 = jnp.concatenate(
        [jnp.sin(argh), jnp.cos(argh), jnp.sin(argw), jnp.cos(argw)], axis=1)
    s_ref[...] = jnp.concatenate(
        [jnp.zeros((hw, 3 * n), jnp.float32), q3], axis=1)


def _stream_kernel(tok_ref, a_ref, s_ref, out_ref):
    out_ref[...] = (tok_ref[...]
                    + a_ref[...][:, None]
                    + s_ref[...][:, :, None, None, :][None])


def kernel(tokens, channel_embed, timestamps, patch_size, input_res):
    b, h, w, t, s, d = tokens.shape
    n = d // 4
    ts = t * s
    hw = h * w

    months = timestamps[:, :, 1].astype(jnp.int32)
    gsd = (jnp.asarray(input_res, jnp.float32)
           * jnp.asarray(patch_size, jnp.float32) / _BASE_GSD).reshape(1, 1)

    a_tab, s_tab = pl.pallas_call(
        functools.partial(_tables_kernel, w_cnt=w),
        grid=(b,),
        in_specs=[
            pl.BlockSpec(memory_space=pltpu.SMEM),
            pl.BlockSpec(memory_space=pltpu.SMEM),
            pl.BlockSpec((channel_embed.shape[0], n), lambda i: (0, 0)),
        ],
        out_specs=[
            pl.BlockSpec((1, ts, d), lambda i: (i, 0, 0)),
            pl.BlockSpec((hw, d), lambda i: (0, 0)),
        ],
        out_shape=[
            jax.ShapeDtypeStruct((b, ts, d), jnp.float32),
            jax.ShapeDtypeStruct((hw, d), jnp.float32),
        ],
    )(months, gsd, channel_embed)

    a_4d = a_tab.reshape(b, t, s, d)
    s_3d = s_tab.reshape(h, w, d)
    out = pl.pallas_call(
        _stream_kernel,
        grid=(b * h,),
        in_specs=[
            pl.BlockSpec((1, 1, w, t, s, d), lambda i: (i // h, i % h, 0, 0, 0, 0)),
            pl.BlockSpec((1, t, s, d), lambda i: (i // h, 0, 0, 0)),
            pl.BlockSpec((1, w, d), lambda i: (i % h, 0, 0)),
        ],
        out_specs=pl.BlockSpec((1, 1, w, t, s, d),
                               lambda i: (i // h, i % h, 0, 0, 0, 0)),
        out_shape=jax.ShapeDtypeStruct(tokens.shape, jnp.float32),
        compiler_params=pltpu.CompilerParams(
            dimension_semantics=("parallel",)),
    )(tokens, a_4d, s_3d)
    return out


# speedup vs baseline: 4.0926x; 1.0337x over previous
import functools
import math

import jax
import jax.numpy as jnp
from jax.experimental import pallas as pl
from jax.experimental.pallas import tpu as pltpu

_BASE_GSD = 10.0
_LN10K = math.log(10000.0)
_MONTH_SCALE = 2.0 * math.pi / 12.0


def _tables_kernel(months_ref, gsd_ref, ce_ref, a_ref, s_ref, *, w_cnt):
    bi = pl.program_id(0)
    ts, d = a_ref.shape[1], a_ref.shape[2]
    n = d // 4
    half = n // 2
    s_cnt = ce_ref.shape[0]
    t_cnt = ts // s_cnt

    r = jax.lax.broadcasted_iota(jnp.int32, (ts, half), 0)
    lane = jax.lax.broadcasted_iota(jnp.int32, (ts, half), 1).astype(jnp.float32)
    t_idx = r // s_cnt
    omega = jnp.exp(lane * (-_LN10K / half))
    arg = t_idx.astype(jnp.float32) * omega
    q1 = jnp.concatenate([jnp.sin(arg), jnp.cos(arg)], axis=1)

    m = jnp.zeros((ts, half), jnp.float32)
    for j in range(t_cnt):
        mj = months_ref[bi, j].astype(jnp.float32)
        m = jnp.where(t_idx == j, mj, m)
    ang = m * _MONTH_SCALE
    q2 = jnp.concatenate([jnp.sin(ang), jnp.cos(ang)], axis=1)

    s_idx = jax.lax.broadcasted_iota(jnp.int32, (ts, n), 0) % s_cnt
    q0 = jnp.zeros((ts, n), jnp.float32)
    for j in range(s_cnt):
        q0 = jnp.where(s_idx == j, ce_ref[j, :][None, :], q0)

    a_ref[0] = jnp.concatenate(
        [q0, q1, q2, jnp.zeros((ts, n), jnp.float32)], axis=1)

    hw = s_ref.shape[0]
    quarter = half // 2
    g = jax.lax.broadcasted_iota(jnp.int32, (hw, quarter), 0)
    lane_q = jax.lax.broadcasted_iota(jnp.int32, (hw, quarter), 1).astype(jnp.float32)
    gsd = gsd_ref[0, 0]
    omega_q = jnp.exp(lane_q * (-_LN10K / quarter))
    ph = (g // w_cnt).astype(jnp.float32) * gsd
    pw = (g % w_cnt).astype(jnp.float32) * gsd
    argh = ph * omega_q
    argw = pw * omega_q
    q3 = jnp.concatenate(
        [jnp.sin(argh), jnp.cos(argh), jnp.sin(argw), jnp.cos(argw)], axis=1)
    s_ref[...] = jnp.concatenate(
        [jnp.zeros((hw, 3 * n), jnp.float32), q3], axis=1)


def _stream_kernel(tok_ref, a_ref, s_ref, out_ref):
    out_ref[...] = (tok_ref[...]
                    + a_ref[...][:, None]
                    + s_ref[...][:, :, None, None, :][None])


def kernel(tokens, channel_embed, timestamps, patch_size, input_res):
    b, h, w, t, s, d = tokens.shape
    n = d // 4
    ts = t * s
    hw = h * w

    months = timestamps[:, :, 1].astype(jnp.int32)
    gsd = (jnp.asarray(input_res, jnp.float32)
           * jnp.asarray(patch_size, jnp.float32) / _BASE_GSD).reshape(1, 1)

    a_tab, s_tab = pl.pallas_call(
        functools.partial(_tables_kernel, w_cnt=w),
        grid=(b,),
        in_specs=[
            pl.BlockSpec(memory_space=pltpu.SMEM),
            pl.BlockSpec(memory_space=pltpu.SMEM),
            pl.BlockSpec((channel_embed.shape[0], n), lambda i: (0, 0)),
        ],
        out_specs=[
            pl.BlockSpec((1, ts, d), lambda i: (i, 0, 0)),
            pl.BlockSpec((hw, d), lambda i: (0, 0)),
        ],
        out_shape=[
            jax.ShapeDtypeStruct((b, ts, d), jnp.float32),
            jax.ShapeDtypeStruct((hw, d), jnp.float32),
        ],
    )(months, gsd, channel_embed)

    h_blk = 2
    nhb = h // h_blk
    a_4d = a_tab.reshape(b, t, s, d)
    s_3d = s_tab.reshape(h, w, d)
    out = pl.pallas_call(
        _stream_kernel,
        grid=(b * nhb,),
        in_specs=[
            pl.BlockSpec((1, h_blk, w, t, s, d),
                         lambda i: (i // nhb, i % nhb, 0, 0, 0, 0)),
            pl.BlockSpec((1, t, s, d), lambda i: (i // nhb, 0, 0, 0)),
            pl.BlockSpec((h_blk, w, d), lambda i: (i % nhb, 0, 0)),
        ],
        out_specs=pl.BlockSpec((1, h_blk, w, t, s, d),
                               lambda i: (i // nhb, i % nhb, 0, 0, 0, 0)),
        out_shape=jax.ShapeDtypeStruct(tokens.shape, jnp.float32),
        compiler_params=pltpu.CompilerParams(
            dimension_semantics=("parallel",)),
    )(tokens, a_4d, s_3d)
    return out


# speedup vs baseline: 4.1469x; 1.0133x over previous
import functools
import math

import jax
import jax.numpy as jnp
from jax.experimental import pallas as pl
from jax.experimental.pallas import tpu as pltpu

_BASE_GSD = 10.0
_LN10K = math.log(10000.0)
_MONTH_SCALE = 2.0 * math.pi / 12.0


def _tables_kernel(months_ref, gsd_ref, ce_ref, a_ref, s_ref, *, w_cnt):
    bi = pl.program_id(0)
    ts, d = a_ref.shape[1], a_ref.shape[2]
    n = d // 4
    half = n // 2
    s_cnt = ce_ref.shape[0]
    t_cnt = ts // s_cnt

    r = jax.lax.broadcasted_iota(jnp.int32, (ts, half), 0)
    lane = jax.lax.broadcasted_iota(jnp.int32, (ts, half), 1).astype(jnp.float32)
    t_idx = r // s_cnt
    omega = jnp.exp(lane * (-_LN10K / half))
    arg = t_idx.astype(jnp.float32) * omega
    q1 = jnp.concatenate([jnp.sin(arg), jnp.cos(arg)], axis=1)

    m = jnp.zeros((ts, half), jnp.float32)
    for j in range(t_cnt):
        mj = months_ref[bi, j].astype(jnp.float32)
        m = jnp.where(t_idx == j, mj, m)
    ang = m * _MONTH_SCALE
    q2 = jnp.concatenate([jnp.sin(ang), jnp.cos(ang)], axis=1)

    s_idx = jax.lax.broadcasted_iota(jnp.int32, (ts, n), 0) % s_cnt
    q0 = jnp.zeros((ts, n), jnp.float32)
    for j in range(s_cnt):
        q0 = jnp.where(s_idx == j, ce_ref[j, :][None, :], q0)

    a_ref[0] = jnp.concatenate(
        [q0, q1, q2, jnp.zeros((ts, n), jnp.float32)], axis=1)

    hw = s_ref.shape[0]
    quarter = half // 2
    g = jax.lax.broadcasted_iota(jnp.int32, (hw, quarter), 0)
    lane_q = jax.lax.broadcasted_iota(jnp.int32, (hw, quarter), 1).astype(jnp.float32)
    gsd = gsd_ref[0, 0]
    omega_q = jnp.exp(lane_q * (-_LN10K / quarter))
    ph = (g // w_cnt).astype(jnp.float32) * gsd
    pw = (g % w_cnt).astype(jnp.float32) * gsd
    argh = ph * omega_q
    argw = pw * omega_q
    q3 = jnp.concatenate(
        [jnp.sin(argh), jnp.cos(argh), jnp.sin(argw), jnp.cos(argw)], axis=1)
    s_ref[...] = jnp.concatenate(
        [jnp.zeros((hw, 3 * n), jnp.float32), q3], axis=1)


def _stream_kernel(tok_ref, a_ref, s_ref, out_ref):
    out_ref[...] = (tok_ref[...]
                    + a_ref[...][:, None]
                    + s_ref[...][:, :, None, None, :][None])


def kernel(tokens, channel_embed, timestamps, patch_size, input_res):
    b, h, w, t, s, d = tokens.shape
    n = d // 4
    ts = t * s
    hw = h * w

    months = timestamps[:, :, 1].astype(jnp.int32)
    gsd = (jnp.asarray(input_res, jnp.float32)
           * jnp.asarray(patch_size, jnp.float32) / _BASE_GSD).reshape(1, 1)

    a_tab, s_tab = pl.pallas_call(
        functools.partial(_tables_kernel, w_cnt=w),
        grid=(b,),
        in_specs=[
            pl.BlockSpec(memory_space=pltpu.SMEM),
            pl.BlockSpec(memory_space=pltpu.SMEM),
            pl.BlockSpec((channel_embed.shape[0], n), lambda i: (0, 0)),
        ],
        out_specs=[
            pl.BlockSpec((1, ts, d), lambda i: (i, 0, 0)),
            pl.BlockSpec((hw, d), lambda i: (0, 0)),
        ],
        out_shape=[
            jax.ShapeDtypeStruct((b, ts, d), jnp.float32),
            jax.ShapeDtypeStruct((hw, d), jnp.float32),
        ],
    )(months, gsd, channel_embed)

    h_blk = 4
    nhb = h // h_blk
    a_4d = a_tab.reshape(b, t, s, d)
    s_3d = s_tab.reshape(h, w, d)
    out = pl.pallas_call(
        _stream_kernel,
        grid=(b * nhb,),
        in_specs=[
            pl.BlockSpec((1, h_blk, w, t, s, d),
                         lambda i: (i // nhb, i % nhb, 0, 0, 0, 0)),
            pl.BlockSpec((1, t, s, d), lambda i: (i // nhb, 0, 0, 0)),
            pl.BlockSpec((h_blk, w, d), lambda i: (i % nhb, 0, 0)),
        ],
        out_specs=pl.BlockSpec((1, h_blk, w, t, s, d),
                               lambda i: (i // nhb, i % nhb, 0, 0, 0, 0)),
        out_shape=jax.ShapeDtypeStruct(tokens.shape, jnp.float32),
        compiler_params=pltpu.CompilerParams(
            dimension_semantics=("parallel",)),
    )(tokens, a_4d, s_3d)
    return out


# speedup vs baseline: 4.2191x; 1.0174x over previous
import functools
import math

import jax
import jax.numpy as jnp
from jax.experimental import pallas as pl
from jax.experimental.pallas import tpu as pltpu

_BASE_GSD = 10.0
_LN10K = math.log(10000.0)
_MONTH_SCALE = 2.0 * math.pi / 12.0


def _tables_kernel(months_ref, gsd_ref, ce_ref, a_ref, s_ref, *, w_cnt):
    bi = pl.program_id(0)
    ts, n3 = a_ref.shape[1], a_ref.shape[2]
    n = n3 // 3
    half = n // 2
    s_cnt = ce_ref.shape[0]
    t_cnt = ts // s_cnt

    r = jax.lax.broadcasted_iota(jnp.int32, (ts, half), 0)
    lane = jax.lax.broadcasted_iota(jnp.int32, (ts, half), 1).astype(jnp.float32)
    t_idx = r // s_cnt
    omega = jnp.exp(lane * (-_LN10K / half))
    arg = t_idx.astype(jnp.float32) * omega
    q1 = jnp.concatenate([jnp.sin(arg), jnp.cos(arg)], axis=1)

    m = jnp.zeros((ts, half), jnp.float32)
    for j in range(t_cnt):
        mj = months_ref[bi, j].astype(jnp.float32)
        m = jnp.where(t_idx == j, mj, m)
    ang = m * _MONTH_SCALE
    q2 = jnp.concatenate([jnp.sin(ang), jnp.cos(ang)], axis=1)

    s_idx = jax.lax.broadcasted_iota(jnp.int32, (ts, n), 0) % s_cnt
    q0 = jnp.zeros((ts, n), jnp.float32)
    for j in range(s_cnt):
        q0 = jnp.where(s_idx == j, ce_ref[j, :][None, :], q0)

    a_ref[0] = jnp.concatenate([q0, q1, q2], axis=1)

    hw = s_ref.shape[0]
    quarter = half // 2
    g = jax.lax.broadcasted_iota(jnp.int32, (hw, quarter), 0)
    lane_q = jax.lax.broadcasted_iota(jnp.int32, (hw, quarter), 1).astype(jnp.float32)
    gsd = gsd_ref[0, 0]
    omega_q = jnp.exp(lane_q * (-_LN10K / quarter))
    ph = (g // w_cnt).astype(jnp.float32) * gsd
    pw = (g % w_cnt).astype(jnp.float32) * gsd
    argh = ph * omega_q
    argw = pw * omega_q
    s_ref[...] = jnp.concatenate(
        [jnp.sin(argh), jnp.cos(argh), jnp.sin(argw), jnp.cos(argw)], axis=1)


def _stream_kernel(tok_ref, a_ref, s_ref, out_ref):
    n3 = a_ref.shape[-1]
    out_ref[..., :n3] = tok_ref[..., :n3] + a_ref[...][:, None]
    out_ref[..., n3:] = (tok_ref[..., n3:]
                         + s_ref[...][:, :, None, None, :][None])


def kernel(tokens, channel_embed, timestamps, patch_size, input_res):
    b, h, w, t, s, d = tokens.shape
    n = d // 4
    ts = t * s
    hw = h * w

    months = timestamps[:, :, 1].astype(jnp.int32)
    gsd = (jnp.asarray(input_res, jnp.float32)
           * jnp.asarray(patch_size, jnp.float32) / _BASE_GSD).reshape(1, 1)

    a_tab, s_tab = pl.pallas_call(
        functools.partial(_tables_kernel, w_cnt=w),
        grid=(b,),
        in_specs=[
            pl.BlockSpec(memory_space=pltpu.SMEM),
            pl.BlockSpec(memory_space=pltpu.SMEM),
            pl.BlockSpec((channel_embed.shape[0], n), lambda i: (0, 0)),
        ],
        out_specs=[
            pl.BlockSpec((1, ts, 3 * n), lambda i: (i, 0, 0)),
            pl.BlockSpec((hw, n), lambda i: (0, 0)),
        ],
        out_shape=[
            jax.ShapeDtypeStruct((b, ts, 3 * n), jnp.float32),
            jax.ShapeDtypeStruct((hw, n), jnp.float32),
        ],
    )(months, gsd, channel_embed)

    h_blk = 4
    nhb = h // h_blk
    a_4d = a_tab.reshape(b, t, s, 3 * n)
    s_3d = s_tab.reshape(h, w, n)
    out = pl.pallas_call(
        _stream_kernel,
        grid=(b * nhb,),
        in_specs=[
            pl.BlockSpec((1, h_blk, w, t, s, d),
                         lambda i: (i // nhb, i % nhb, 0, 0, 0, 0)),
            pl.BlockSpec((1, t, s, 3 * n), lambda i: (i // nhb, 0, 0, 0)),
            pl.BlockSpec((h_blk, w, n), lambda i: (i % nhb, 0, 0)),
        ],
        out_specs=pl.BlockSpec((1, h_blk, w, t, s, d),
                               lambda i: (i // nhb, i % nhb, 0, 0, 0, 0)),
        out_shape=jax.ShapeDtypeStruct(tokens.shape, jnp.float32),
        compiler_params=pltpu.CompilerParams(
            dimension_semantics=("parallel",)),
    )(tokens, a_4d, s_3d)
    return out
